# baseline (speedup 1.0000x reference)
import functools

import jax
import jax.numpy as jnp
from jax.experimental import pallas as pl
from jax.experimental.pallas import tpu as pltpu

NUM_EXPERTS = 8
TOP_K = 2
RANK = 128
D_IN = 2048
D_OUT = 2048
ER = NUM_EXPERTS * RANK
BLK = 512


def _moe_body(x_ref, wgn_ref, wd_ref, wu_ref, noise_ref, bexp_ref,
              out_ref, rl_ref, wdb_ref, wub_ref):
    @pl.when(pl.program_id(0) == 0)
    def _prep():
        wdb_ref[...] = wd_ref[...].astype(jnp.bfloat16)
        for e in range(NUM_EXPERTS):
            wub_ref[pl.ds(e * RANK, RANK), :] = (
                wu_ref[e].T.astype(jnp.bfloat16))

    x = x_ref[...]

    ln = jax.lax.dot_general(
        wgn_ref[...], x, (((1,), (1,)), ((), ())),
        preferred_element_type=jnp.float32)
    rl = ln[:NUM_EXPERTS, :] + noise_ref[...].T * jax.nn.softplus(
        ln[NUM_EXPERTS:, :])
    rl_ref[...] = rl.T

    row = jax.lax.broadcasted_iota(jnp.int32, rl.shape, 0)
    m1 = jnp.max(rl, axis=0, keepdims=True)
    a1 = jnp.min(jnp.where(rl == m1, row, NUM_EXPERTS), axis=0, keepdims=True)
    first = row == a1
    rl_m = jnp.where(first, -jnp.inf, rl)
    m2 = jnp.max(rl_m, axis=0, keepdims=True)
    a2 = jnp.min(jnp.where(rl_m == m2, row, NUM_EXPERTS), axis=0,
                 keepdims=True)
    s = 1.0 / (1.0 + jnp.exp(m2 - m1))
    w = jnp.where(first, s, 0.0) + jnp.where(row == a2, 1.0 - s, 0.0)

    xb = x.astype(jnp.bfloat16)
    down = jax.lax.dot_general(
        xb, wdb_ref[...], (((1,), (1,)), ((), ())),
        preferred_element_type=jnp.float32)
    wexp = jax.lax.dot_general(
        w, bexp_ref[...], (((0,), (0,)), ((), ())),
        preferred_element_type=jnp.float32)
    scaled = (down * wexp).astype(jnp.bfloat16)
    up = jax.lax.dot_general(
        scaled, wub_ref[...], (((1,), (0,)), ((), ())),
        preferred_element_type=jnp.float32)
    out_ref[...] = up


@functools.partial(jax.jit, static_argnames=("interpret",))
def kernel(hidden_states, Wg, Wn, W_down, W_up, interpret=False):
    b, s, d = hidden_states.shape
    T = b * s
    x = hidden_states.reshape(T, d)
    noise = jax.random.normal(jax.random.key(42), (T, NUM_EXPERTS),
                              jnp.float32)
    wgn = jnp.concatenate([Wg, Wn], axis=0)
    wd_all = W_down.reshape(ER, D_IN)
    bexp = jnp.repeat(jnp.eye(NUM_EXPERTS, dtype=jnp.float32), RANK, axis=1)

    grid = (T // BLK,)
    out, rl = pl.pallas_call(
        _moe_body,
        grid=grid,
        in_specs=[
            pl.BlockSpec((BLK, D_IN), lambda i: (i, 0)),
            pl.BlockSpec((2 * NUM_EXPERTS, D_IN), lambda i: (0, 0)),
            pl.BlockSpec((ER, D_IN), lambda i: (0, 0)),
            pl.BlockSpec((NUM_EXPERTS, D_OUT, RANK), lambda i: (0, 0, 0)),
            pl.BlockSpec((BLK, NUM_EXPERTS), lambda i: (i, 0)),
            pl.BlockSpec((NUM_EXPERTS, ER), lambda i: (0, 0)),
        ],
        out_specs=[
            pl.BlockSpec((BLK, D_OUT), lambda i: (i, 0)),
            pl.BlockSpec((BLK, NUM_EXPERTS), lambda i: (i, 0)),
        ],
        out_shape=[
            jax.ShapeDtypeStruct((T, D_OUT), jnp.float32),
            jax.ShapeDtypeStruct((T, NUM_EXPERTS), jnp.float32),
        ],
        scratch_shapes=[
            pltpu.VMEM((ER, D_IN), jnp.bfloat16),
            pltpu.VMEM((ER, D_OUT), jnp.bfloat16),
        ],
        compiler_params=pltpu.CompilerParams(
            dimension_semantics=("arbitrary",),
        ),
        interpret=interpret,
    )(x, wgn, wd_all, W_up, noise, bexp)
    return out.reshape(b, s, D_OUT), rl


# speedup vs baseline: 1.2285x; 1.2285x over previous
import functools

import jax
import jax.numpy as jnp
from jax.experimental import pallas as pl
from jax.experimental.pallas import tpu as pltpu

NUM_EXPERTS = 8
TOP_K = 2
RANK = 128
D_IN = 2048
D_OUT = 2048
ER = NUM_EXPERTS * RANK
BLK = 512


def _moe_body(x_ref, wgn_ref, wd_ref, wu_ref, noise_ref, bexp_ref,
              out_ref, rl_ref, wdb_ref, wub_ref):
    @pl.when(pl.program_id(0) == 0)
    def _prep():
        wdb_ref[...] = wd_ref[...].astype(jnp.bfloat16)
        for e in range(NUM_EXPERTS):
            wub_ref[pl.ds(e * RANK, RANK), :] = (
                wu_ref[e].T.astype(jnp.bfloat16))

    x = x_ref[...]

    ln = jax.lax.dot_general(
        wgn_ref[...], x, (((1,), (1,)), ((), ())),
        preferred_element_type=jnp.float32)
    rl = ln[:NUM_EXPERTS, :] + noise_ref[...] * jax.nn.softplus(
        ln[NUM_EXPERTS:, :])
    rl_ref[...] = rl.T

    row = jax.lax.broadcasted_iota(jnp.int32, rl.shape, 0)
    m1 = jnp.max(rl, axis=0, keepdims=True)
    a1 = jnp.min(jnp.where(rl == m1, row, NUM_EXPERTS), axis=0, keepdims=True)
    first = row == a1
    rl_m = jnp.where(first, -jnp.inf, rl)
    m2 = jnp.max(rl_m, axis=0, keepdims=True)
    a2 = jnp.min(jnp.where(rl_m == m2, row, NUM_EXPERTS), axis=0,
                 keepdims=True)
    s = 1.0 / (1.0 + jnp.exp(m2 - m1))
    w = jnp.where(first, s, 0.0) + jnp.where(row == a2, 1.0 - s, 0.0)

    xb = x.astype(jnp.bfloat16)
    down = jax.lax.dot_general(
        xb, wdb_ref[...], (((1,), (1,)), ((), ())),
        preferred_element_type=jnp.float32)
    wexp = jax.lax.dot_general(
        w, bexp_ref[...], (((0,), (0,)), ((), ())),
        preferred_element_type=jnp.float32)
    scaled = (down * wexp).astype(jnp.bfloat16)
    up = jax.lax.dot_general(
        scaled, wub_ref[...], (((1,), (0,)), ((), ())),
        preferred_element_type=jnp.float32)
    out_ref[...] = up


@functools.partial(jax.jit, static_argnames=("interpret",))
def kernel(hidden_states, Wg, Wn, W_down, W_up, interpret=False):
    b, s, d = hidden_states.shape
    T = b * s
    x = hidden_states.reshape(T, d)
    noise = jax.random.normal(jax.random.key(42), (T, NUM_EXPERTS),
                              jnp.float32)
    noise_t = noise.T
    wgn = jnp.concatenate([Wg, Wn], axis=0)
    wd_all = W_down.reshape(ER, D_IN)
    bexp = jnp.repeat(jnp.eye(NUM_EXPERTS, dtype=jnp.float32), RANK, axis=1)

    grid = (T // BLK,)
    out, rl = pl.pallas_call(
        _moe_body,
        grid=grid,
        in_specs=[
            pl.BlockSpec((BLK, D_IN), lambda i: (i, 0)),
            pl.BlockSpec((2 * NUM_EXPERTS, D_IN), lambda i: (0, 0)),
            pl.BlockSpec((ER, D_IN), lambda i: (0, 0)),
            pl.BlockSpec((NUM_EXPERTS, D_OUT, RANK), lambda i: (0, 0, 0)),
            pl.BlockSpec((NUM_EXPERTS, BLK), lambda i: (0, i)),
            pl.BlockSpec((NUM_EXPERTS, ER), lambda i: (0, 0)),
        ],
        out_specs=[
            pl.BlockSpec((BLK, D_OUT), lambda i: (i, 0)),
            pl.BlockSpec((BLK, NUM_EXPERTS), lambda i: (i, 0)),
        ],
        out_shape=[
            jax.ShapeDtypeStruct((T, D_OUT), jnp.float32),
            jax.ShapeDtypeStruct((T, NUM_EXPERTS), jnp.float32),
        ],
        scratch_shapes=[
            pltpu.VMEM((ER, D_IN), jnp.bfloat16),
            pltpu.VMEM((ER, D_OUT), jnp.bfloat16),
        ],
        compiler_params=pltpu.CompilerParams(
            dimension_semantics=("arbitrary",),
        ),
        interpret=interpret,
    )(x, wgn, wd_all, W_up, noise_t, bexp)
    return out.reshape(b, s, D_OUT), rl
